# baseline (speedup 1.0000x reference)
import functools
import math

import jax
import jax.numpy as jnp
from jax import lax
from jax.experimental import pallas as pl
from jax.experimental.pallas import tpu as pltpu
from jax.experimental.pallas import tpu_sc as plsc

N_RAYS = 16384
N_SAMPLES = 524288
CX = 64
FAR_C, FAR_H, FAR_L = 32, 128, 4
NEAR_C, NEAR_H, NEAR_L = 16, 256, 4
T_CONST = 0.75
INV_LN2 = 1.4426950408889634

NC, NS = 2, 16
NW = NC * NS


def _softplus(x):
    m = jnp.maximum(x, 0.0)
    return m + jnp.log(jnp.exp(x - m) + jnp.exp(-m))


def _mlp3(x, p):
    (w1, b1), (w2, b2), (w3, b3) = p
    h = jnp.maximum(jnp.dot(x, w1, preferred_element_type=jnp.float32) + b1, 0.0)
    h = jnp.maximum(jnp.dot(h, w2, preferred_element_type=jnp.float32) + b2, 0.0)
    return jnp.dot(h, w3, preferred_element_type=jnp.float32) + b3


_FUSE_B = 4096
_N_TROWS = 3 * NEAR_L * NEAR_H * NEAR_H


def _fuse_kernel(a_ref, b_ref, o_ref):
    o_ref[:, :NEAR_C] = a_ref[...].astype(jnp.bfloat16)
    o_ref[:, NEAR_C:] = b_ref[...].astype(jnp.bfloat16)


def _fuse_tables(tri_n, tri_s):
    grid = _N_TROWS // _FUSE_B
    return pl.pallas_call(
        _fuse_kernel,
        grid=(grid,),
        in_specs=[pl.BlockSpec((_FUSE_B, NEAR_C), lambda i: (i, 0)),
                  pl.BlockSpec((_FUSE_B, NEAR_C), lambda i: (i, 0))],
        out_specs=pl.BlockSpec((_FUSE_B, 2 * NEAR_C), lambda i: (i, 0)),
        out_shape=jax.ShapeDtypeStruct((_N_TROWS, 2 * NEAR_C), jnp.bfloat16),
    )(tri_n, tri_s)


def _bilinear_corners(u, v, h):
    gx = jnp.clip(u, 0.0, 1.0) * (h - 1)
    gy = jnp.clip(v, 0.0, 1.0) * (h - 1)
    x0f = jnp.floor(gx)
    y0f = jnp.floor(gy)
    x0 = x0f.astype(jnp.int32)
    y0 = y0f.astype(jnp.int32)
    x1 = jnp.minimum(x0 + 1, h - 1)
    y1 = jnp.minimum(y0 + 1, h - 1)
    wx = gx - x0f
    wy = gy - y0f
    return x0, x1, y0, y1, wx, wy


def _mip_levels(r, h, l):
    lvl = jnp.clip(jnp.log(jnp.maximum(r, 1e-6) * h) * INV_LN2, 0.0, l - 1.0)
    l0f = jnp.floor(lvl)
    l0 = l0f.astype(jnp.int32)
    l1 = jnp.minimum(l0 + 1, l - 1)
    wl = lvl - l0f
    return l0, l1, wl


def _corner8(base0, base1, wl, x0, x1, y0, y1, wx, wy, h):
    idxs, ws = [], []
    for lb, lw in ((base0, 1.0 - wl), (base1, wl)):
        for yy, wyy in ((y0, 1.0 - wy), (y1, wy)):
            for xx, wxx in ((x0, 1.0 - wx), (x1, wx)):
                idxs.append(lb + yy * h + xx)
                ws.append(lw * wyy * wxx)
    return idxs, ws


def _prep_rays_kernel(xt_ref, wit_ref, rg_ref, oidx_ref, ow_ref, otab_ref):
    d0 = (wit_ref[0] * 0.5 + 0.5) * 2.0 - 1.0
    d1 = (wit_ref[1] * 0.5 + 0.5) * 2.0 - 1.0
    d2 = (wit_ref[2] * 0.5 + 0.5) * 2.0 - 1.0
    a0, a1, a2 = jnp.abs(d0), jnp.abs(d1), jnp.abs(d2)
    ax0 = (a0 >= a1) & (a0 >= a2)
    ax1 = (~ax0) & (a1 >= a2)
    maj = jnp.where(ax0, d0, jnp.where(ax1, d1, d2))
    face = (jnp.where(ax0, 0, jnp.where(ax1, 2, 4))
            + (maj < 0).astype(jnp.int32))
    su = jnp.where(ax0, d1, d0)
    sv = jnp.where(ax0 | ax1, d2, d1)
    den = jnp.maximum(jnp.abs(maj), 1e-6)
    u = (su / den) * 0.5 + 0.5
    v = (sv / den) * 0.5 + 0.5
    r = rg_ref[0]
    l0, l1, wl = _mip_levels(r, FAR_H, FAR_L)
    x0, x1, y0, y1, wx, wy = _bilinear_corners(u, v, FAR_H)
    fb = face * (FAR_L * FAR_H * FAR_H)
    hh = FAR_H * FAR_H
    idxs, ws = _corner8(fb + l0 * hh, fb + l1 * hh, wl,
                        x0, x1, y0, y1, wx, wy, FAR_H)
    oidx_ref[...] = jnp.concatenate([i[None] for i in idxs], axis=0)
    ow_ref[...] = jnp.concatenate([w[None] for w in ws], axis=0)
    r0 = r * r * math.sqrt(T_CONST / (1.0 - T_CONST))
    zero = jnp.zeros_like(r0)
    rows = [xt_ref[0], xt_ref[1], xt_ref[2],
            wit_ref[0], wit_ref[1], wit_ref[2], r0] + [zero] * 9
    otab_ref[...] = jnp.concatenate([q[None] for q in rows], axis=0)


def _prep_rays(xt, wit, rg):
    return pl.pallas_call(
        _prep_rays_kernel,
        out_shape=(jax.ShapeDtypeStruct((8, N_RAYS), jnp.int32),
                   jax.ShapeDtypeStruct((8, N_RAYS), jnp.float32),
                   jax.ShapeDtypeStruct((16, N_RAYS), jnp.float32)),
    )(xt, wit, rg)


def _sc_gather1(ray_tab, ray_idx2d, cub_idx2d, cub_tab):
    mesh = plsc.VectorSubcoreMesh(core_axis_name="c", subcore_axis_name="s",
                                  num_cores=NC, num_subcores=NS)
    n_attr_rows = N_SAMPLES // NW
    n_cub_rows = N_RAYS * 8 // NW
    CH = 2048
    G = CH // 128

    @functools.partial(
        pl.kernel, mesh=mesh,
        compiler_params=pltpu.CompilerParams(use_tc_tiling_on_sc=False),
        out_type=(jax.ShapeDtypeStruct((N_SAMPLES, 16), jnp.float32),
                  jax.ShapeDtypeStruct((N_RAYS * 8, FAR_C), jnp.float32)),
        scratch_types=[pltpu.VMEM((G, 128), jnp.int32),
                       pltpu.VMEM((CH, 16), jnp.float32),
                       pltpu.VMEM((CH, FAR_C), jnp.float32),
                       pltpu.SemaphoreType.DMA],
    )
    def k(tab_hbm, ridx_hbm, cidx_hbm, ctab_hbm, attrs_hbm, crows_hbm,
          idx_v, rows_v, crows_v, sem):
        wid = lax.axis_index("s") * NC + lax.axis_index("c")

        @pl.loop(0, n_attr_rows // CH)
        def _(i):
            base = wid * n_attr_rows + i * CH
            pltpu.sync_copy(
                ridx_hbm.at[pl.ds(wid * (n_attr_rows // 128) + i * G, G)],
                idx_v)
            for g in range(G):
                pltpu.async_copy(
                    tab_hbm.at[idx_v.at[g]],
                    rows_v.at[pl.ds(g * 128, 128)], sem)
            for g in range(G):
                pltpu.make_async_copy(
                    tab_hbm.at[idx_v.at[g]],
                    rows_v.at[pl.ds(g * 128, 128)], sem).wait()
            pltpu.sync_copy(rows_v, attrs_hbm.at[pl.ds(base, CH)])

        @pl.loop(0, n_cub_rows // CH)
        def _(i):
            base = wid * n_cub_rows + i * CH
            pltpu.sync_copy(
                cidx_hbm.at[pl.ds(wid * (n_cub_rows // 128) + i * G, G)],
                idx_v)
            for g in range(G):
                pltpu.async_copy(
                    ctab_hbm.at[idx_v.at[g]],
                    crows_v.at[pl.ds(g * 128, 128)], sem)
            for g in range(G):
                pltpu.make_async_copy(
                    ctab_hbm.at[idx_v.at[g]],
                    crows_v.at[pl.ds(g * 128, 128)], sem).wait()
            pltpu.sync_copy(crows_v, crows_hbm.at[pl.ds(base, CH)])

    return k(ray_tab, ray_idx2d, cub_idx2d, cub_tab)


_SP_R = 32
_SP_BLK = _SP_R * 128
_SP_GRID = N_SAMPLES // _SP_BLK


def _prep_samples_kernel(at_ref, ts_ref, te_ref, oidx_ref, ow_ref, ov_ref):
    at = at_ref[...]
    ts, te = ts_ref[...], te_ref[...]
    tm = 0.5 * (ts + te)
    xn = [(at[j] + tm * at[3 + j] + 1.0) * 0.5 for j in range(3)]
    rn = at[6] * tm * 0.5
    valid = jnp.ones_like(tm)
    for q in xn:
        valid = valid * ((q >= 0.0) & (q <= 1.0)).astype(jnp.float32)
    ov_ref[...] = valid
    l0, l1, wl = _mip_levels(rn, NEAR_H, NEAR_L)
    hh = NEAR_H * NEAR_H
    idxs_all, ws_all = [], []
    for p, (a, b) in enumerate(((0, 1), (0, 2), (1, 2))):
        x0, x1, y0, y1, wx, wy = _bilinear_corners(xn[a], xn[b], NEAR_H)
        pb = p * NEAR_L * hh
        idxs, ws = _corner8(pb + l0 * hh, pb + l1 * hh, wl,
                            x0, x1, y0, y1, wx, wy, NEAR_H)
        idxs_all += idxs
        ws_all += ws
    oidx_ref[...] = jnp.concatenate([q[None] for q in idxs_all], axis=0)
    ow_ref[...] = jnp.concatenate([q[None] for q in ws_all], axis=0)


def _prep_samples(attrs_t, tsr, ter):
    return pl.pallas_call(
        _prep_samples_kernel,
        grid=(_SP_GRID,),
        in_specs=[pl.BlockSpec((16, _SP_R, 128), lambda i: (0, i, 0)),
                  pl.BlockSpec((_SP_R, 128), lambda i: (i, 0)),
                  pl.BlockSpec((_SP_R, 128), lambda i: (i, 0))],
        out_specs=(pl.BlockSpec((24, _SP_R, 128), lambda i: (0, i, 0)),
                   pl.BlockSpec((24, _SP_R, 128), lambda i: (0, i, 0)),
                   pl.BlockSpec((_SP_R, 128), lambda i: (i, 0))),
        out_shape=(jax.ShapeDtypeStruct((24, N_SAMPLES // 128, 128), jnp.int32),
                   jax.ShapeDtypeStruct((24, N_SAMPLES // 128, 128),
                                        jnp.float32),
                   jax.ShapeDtypeStruct((N_SAMPLES // 128, 128), jnp.float32)),
    )(attrs_t, tsr, ter)


_NG_ROWS = N_SAMPLES * 24


def _sc_gather2(fused, idx2d):
    mesh = plsc.VectorSubcoreMesh(core_axis_name="c", subcore_axis_name="s",
                                  num_cores=NC, num_subcores=NS)
    per_w = _NG_ROWS // NW
    CH = 1024
    G = CH // 128

    @functools.partial(
        pl.kernel, mesh=mesh,
        compiler_params=pltpu.CompilerParams(use_tc_tiling_on_sc=False),
        out_type=jax.ShapeDtypeStruct((_NG_ROWS, 2 * NEAR_C), jnp.bfloat16),
        scratch_types=[pltpu.VMEM((G, 128), jnp.int32),
                       pltpu.VMEM((CH, 2 * NEAR_C), jnp.bfloat16),
                       pltpu.VMEM((G, 128), jnp.int32),
                       pltpu.VMEM((CH, 2 * NEAR_C), jnp.bfloat16),
                       pltpu.SemaphoreType.DMA,
                       pltpu.SemaphoreType.DMA],
    )
    def k(tab_hbm, idx_hbm, g_hbm, idx_a, rows_a, idx_b, rows_b, sem_a, sem_b):
        wid = lax.axis_index("s") * NC + lax.axis_index("c")
        base_w = wid * per_w

        def fire(idx_v, rows_v, sem, i):
            pltpu.sync_copy(
                idx_hbm.at[pl.ds(wid * (per_w // 128) + i * G, G)], idx_v)
            for g in range(G):
                pltpu.async_copy(tab_hbm.at[idx_v.at[g]],
                                 rows_v.at[pl.ds(g * 128, 128)], sem)

        def drain_store(idx_v, rows_v, sem, i):
            base = base_w + i * CH
            for g in range(G):
                pltpu.make_async_copy(tab_hbm.at[idx_v.at[g]],
                                      rows_v.at[pl.ds(g * 128, 128)],
                                      sem).wait()
            pltpu.sync_copy(rows_v, g_hbm.at[pl.ds(base, CH)])

        n_ch = per_w // CH
        fire(idx_a, rows_a, sem_a, 0)

        @pl.loop(0, n_ch // 2)
        def _(j):
            fire(idx_b, rows_b, sem_b, 2 * j + 1)
            drain_store(idx_a, rows_a, sem_a, 2 * j)

            @pl.when(2 * j + 2 < n_ch)
            def _():
                fire(idx_a, rows_a, sem_a, 2 * j + 2)
            drain_store(idx_b, rows_b, sem_b, 2 * j + 1)

    return k(fused, idx2d)


_MB = 512
_M_GRID = N_SAMPLES // _MB


def _main_kernel(g_ref, w_ref, v_ref,
                 ws1_ref, bs1_ref, ws2_ref, bs2_ref, ws3_ref, bs3_ref,
                 wn1_ref, bn1_ref, wn2_ref, bn2_ref, wn3_ref, bn3_ref,
                 sig_ref, hn_ref):
    g = g_ref[...].astype(jnp.float32)
    w = w_ref[...]
    gw = g * w
    s0 = jnp.sum(gw[0:8], axis=0)
    s1 = jnp.sum(gw[8:16], axis=0)
    s2 = jnp.sum(gw[16:24], axis=0)
    feats_n = jnp.concatenate(
        [s0[:, :NEAR_C], s1[:, :NEAR_C], s2[:, :NEAR_C]], axis=1)
    feats_s = jnp.concatenate(
        [s0[:, NEAR_C:], s1[:, NEAR_C:], s2[:, NEAR_C:]], axis=1)
    ps = ((ws1_ref[...], bs1_ref[...]), (ws2_ref[...], bs2_ref[...]),
          (ws3_ref[...], bs3_ref[...]))
    pn = ((wn1_ref[...], bn1_ref[...]), (wn2_ref[...], bn2_ref[...]),
          (wn3_ref[...], bn3_ref[...]))
    sig = _mlp3(feats_s, ps)
    sig_ref[...] = _softplus(sig) * v_ref[...]
    hn_ref[...] = _mlp3(feats_n, pn)


def _tc_main(g3, w3, valid_col, p_s, p_n):
    full = lambda s: pl.BlockSpec(s, lambda i: tuple(0 for _ in s))
    params = []
    specs = []
    for (w1, b1), (w2, b2), (w3_, b3) in (p_s, p_n):
        for arr in (w1, b1.reshape(1, -1), w2, b2.reshape(1, -1),
                    w3_, b3.reshape(1, -1)):
            params.append(arr)
            specs.append(full(arr.shape))
    return pl.pallas_call(
        _main_kernel,
        grid=(_M_GRID,),
        in_specs=[pl.BlockSpec((24, _MB, 2 * NEAR_C), lambda i: (0, i, 0)),
                  pl.BlockSpec((24, _MB, 1), lambda i: (0, i, 0)),
                  pl.BlockSpec((_MB, 1), lambda i: (i, 0))] + specs,
        out_specs=(pl.BlockSpec((_MB, 1), lambda i: (i, 0)),
                   pl.BlockSpec((_MB, FAR_C), lambda i: (i, 0))),
        out_shape=(jax.ShapeDtypeStruct((N_SAMPLES, 1), jnp.float32),
                   jax.ShapeDtypeStruct((N_SAMPLES, FAR_C), jnp.float32)),
    )(g3, w3, valid_col, *params)


_SC_R = N_SAMPLES // 128


def _shift_lanes(y, s, fill=0.0):
    pad = jnp.full((y.shape[0], s), fill, y.dtype)
    return jnp.concatenate([pad, y[:, :-s]], axis=1)


def _shift_rows(y, s, fill=0.0):
    pad = jnp.full((s, y.shape[1]), fill, y.dtype)
    return jnp.concatenate([pad, y[:-s]], axis=0)


def _flat_scan(x, op):
    c = x
    s = 1
    while s < 128:
        c = op(c, _shift_lanes(c, s))
        s *= 2
    t = jnp.broadcast_to(c[:, 127:128], c.shape)
    s = 1
    while s < _SC_R:
        t = op(t, _shift_rows(t, s))
        s *= 2
    return op(c, _shift_rows(t, 1))


def _scan_kernel(sig_ref, ts_ref, te_ref, ray_ref, w_ref):
    sig = sig_ref[...]
    dt = te_ref[...] - ts_ref[...]
    sdt = sig * dt
    cum = _flat_scan(sdt, jnp.add)
    excl = cum - sdt
    ray = ray_ref[...]
    prev = _shift_lanes(ray, 1, 0)
    prev_row = _shift_rows(ray[:, 127:128], 1, -1)
    lane0 = lax.broadcasted_iota(jnp.int32, ray.shape, 1) == 0
    prev = jnp.where(lane0, jnp.broadcast_to(prev_row, ray.shape), prev)
    is_start = ray != prev
    m = jnp.where(is_start, excl, 0.0)
    seg_first = _flat_scan(m, jnp.maximum)
    alpha = 1.0 - jnp.exp(-sdt)
    trans = jnp.exp(-(excl - seg_first))
    w_ref[...] = trans * alpha


def _tc_scan(sig_r, tsr, ter, rayr):
    return pl.pallas_call(
        _scan_kernel,
        out_shape=jax.ShapeDtypeStruct((_SC_R, 128), jnp.float32),
    )(sig_r, tsr, ter, rayr)


def _rows_kernel(hn_ref, w_ref, o_ref):
    w = w_ref[...]
    o_ref[...] = jnp.concatenate(
        [hn_ref[...] * w, w, jnp.zeros((w.shape[0], 15), jnp.float32)], axis=1)


def _tc_rows(h_n, w_col):
    return pl.pallas_call(
        _rows_kernel,
        grid=(_M_GRID,),
        in_specs=[pl.BlockSpec((_MB, FAR_C), lambda i: (i, 0)),
                  pl.BlockSpec((_MB, 1), lambda i: (i, 0))],
        out_specs=pl.BlockSpec((_MB, 48), lambda i: (i, 0)),
        out_shape=jax.ShapeDtypeStruct((N_SAMPLES, 48), jnp.float32),
    )(h_n, w_col)


def _sc_scatter(rows, ray_idx2d, zeros_init):
    mesh = plsc.VectorSubcoreMesh(core_axis_name="c", subcore_axis_name="s",
                                  num_cores=NC, num_subcores=NS)
    per_w = N_SAMPLES // NW
    CH = 1024
    G = CH // 128
    rows_per_tile = N_RAYS // NS

    @functools.partial(
        pl.kernel, mesh=mesh,
        compiler_params=pltpu.CompilerParams(use_tc_tiling_on_sc=False),
        out_type=jax.ShapeDtypeStruct((NC, N_RAYS, 48), jnp.float32),
        scratch_types=[pltpu.VMEM((G, 128), jnp.int32),
                       pltpu.VMEM((CH, 48), jnp.float32),
                       pltpu.VMEM_SHARED((N_RAYS, 48), jnp.float32)],
    )
    def k(rows_hbm, ridx_hbm, zer_hbm, out_hbm, idx_v, rows_v, acc_sh):
        cid = lax.axis_index("c")
        sid = lax.axis_index("s")
        pltpu.sync_copy(zer_hbm.at[pl.ds(sid * rows_per_tile, rows_per_tile)],
                        acc_sh.at[pl.ds(sid * rows_per_tile, rows_per_tile)])
        plsc.subcore_barrier()
        wid = sid * NC + cid

        @pl.loop(0, per_w // CH)
        def _(i):
            base = wid * per_w + i * CH
            pltpu.sync_copy(
                ridx_hbm.at[pl.ds(wid * (per_w // 128) + i * G, G)], idx_v)
            pltpu.sync_copy(rows_hbm.at[pl.ds(base, CH)], rows_v)
            for g in range(G):
                pltpu.sync_copy(rows_v.at[pl.ds(g * 128, 128)],
                                acc_sh.at[idx_v.at[g]], add=True)
        plsc.subcore_barrier()
        pltpu.sync_copy(acc_sh.at[pl.ds(sid * rows_per_tile, rows_per_tile)],
                        out_hbm.at[cid, pl.ds(sid * rows_per_tile,
                                              rows_per_tile)])

    return k(rows, ray_idx2d, zeros_init)


_FB = 2048
_F_GRID = N_RAYS // _FB


def _final_kernel(p_ref, cr_ref, cw_ref, fx_ref, wo_ref,
                  w1_ref, b1_ref, w2_ref, b2_ref, w3_ref, b3_ref, o_ref):
    p = p_ref[...]
    acc = p[0] + p[1]
    h_acc = acc[:, :FAR_C]
    alpha_n = acc[:, FAR_C:FAR_C + 1]
    cw = cw_ref[...]
    h_f = jnp.zeros((_FB, FAR_C), jnp.float32)
    for j in range(8):
        h_f = h_f + cw[:, j:j + 1] * cr_ref[j]
    h = h_f * (1.0 - alpha_n) + h_acc
    inp = jnp.concatenate([fx_ref[...], h, wo_ref[...]], axis=1)
    pd = ((w1_ref[...], b1_ref[...]), (w2_ref[...], b2_ref[...]),
          (w3_ref[...], b3_ref[...]))
    out = _mlp3(inp, pd)
    o_ref[...] = 1.0 / (1.0 + jnp.exp(-out))


def _tc_final(partials, cub_rows3, cub_w, fx, wo_o_n, p_d):
    full = lambda s: pl.BlockSpec(s, lambda i: tuple(0 for _ in s))
    params, specs = [], []
    for w_, b_ in p_d:
        for arr in (w_, b_.reshape(1, -1)):
            params.append(arr)
            specs.append(full(arr.shape))
    return pl.pallas_call(
        _final_kernel,
        grid=(_F_GRID,),
        in_specs=[pl.BlockSpec((NC, _FB, 48), lambda i: (0, i, 0)),
                  pl.BlockSpec((8, _FB, FAR_C), lambda i: (0, i, 0)),
                  pl.BlockSpec((_FB, 8), lambda i: (i, 0)),
                  pl.BlockSpec((_FB, CX), lambda i: (i, 0)),
                  pl.BlockSpec((_FB, 1), lambda i: (i, 0))] + specs,
        out_specs=pl.BlockSpec((_FB, 3), lambda i: (i, 0)),
        out_shape=jax.ShapeDtypeStruct((N_RAYS, 3), jnp.float32),
    )(partials, cub_rows3, cub_w, fx, wo_o_n, *params)


def kernel(x, wi, roughness, fx, wo_o_n, ray_indices, t_starts, t_ends,
           params):
    cub_tab = params['cubemap'].reshape(6 * FAR_L * FAR_H * FAR_H, FAR_C)
    tri_n = params['tri_n'].reshape(_N_TROWS, NEAR_C)
    tri_s = params['tri_n_sigma'].reshape(_N_TROWS, NEAR_C)
    fused = _fuse_tables(tri_n, tri_s)

    cub_idx, cub_w, ray_tab = _prep_rays(
        x.T, wi.T, roughness.reshape(1, N_RAYS))
    ray_tab_r = ray_tab.T.reshape(N_RAYS, 16)
    cub_idx_flat = cub_idx.reshape(N_RAYS * 8 // 128, 128)
    cub_w_r = cub_w.T

    ray_idx2d = ray_indices.reshape(N_SAMPLES // 128, 128)
    attrs, cub_rows = _sc_gather1(ray_tab_r, ray_idx2d, cub_idx_flat, cub_tab)

    attrs_t = attrs.T.reshape(16, N_SAMPLES // 128, 128)
    tsr = t_starts.reshape(N_SAMPLES // 128, 128)
    ter = t_ends.reshape(N_SAMPLES // 128, 128)
    idx24, w24, valid = _prep_samples(attrs_t, tsr, ter)

    idx_flat = idx24.reshape(_NG_ROWS // 128, 128)
    g = _sc_gather2(fused, idx_flat)

    g3 = g.reshape(24, N_SAMPLES, 2 * NEAR_C)
    w3 = w24.reshape(24, N_SAMPLES, 1)
    valid_col = valid.reshape(N_SAMPLES, 1)
    sig, h_n = _tc_main(g3, w3, valid_col,
                        params['mlp_n_sigma'], params['mlp_n'])

    w_flat = _tc_scan(sig.reshape(N_SAMPLES // 128, 128), tsr, ter,
                      ray_indices.reshape(N_SAMPLES // 128, 128))
    rows = _tc_rows(h_n, w_flat.reshape(N_SAMPLES, 1))

    zeros_init = jnp.zeros((N_RAYS, 48), jnp.float32)
    partials = _sc_scatter(rows, ray_idx2d, zeros_init)

    cub_rows3 = cub_rows.reshape(8, N_RAYS, FAR_C)
    return _tc_final(partials, cub_rows3, cub_w_r, fx, wo_o_n,
                     params['mlp_d'])


# speedup vs baseline: 1.5796x; 1.5796x over previous
import functools
import math

import jax
import jax.numpy as jnp
from jax import lax
from jax.experimental import pallas as pl
from jax.experimental.pallas import tpu as pltpu
from jax.experimental.pallas import tpu_sc as plsc

N_RAYS = 16384
N_SAMPLES = 524288
CX = 64
FAR_C, FAR_H, FAR_L = 32, 128, 4
NEAR_C, NEAR_H, NEAR_L = 16, 256, 4
T_CONST = 0.75
INV_LN2 = 1.4426950408889634

NC, NS = 2, 16
NW = NC * NS


def _softplus(x):
    m = jnp.maximum(x, 0.0)
    return m + jnp.log(jnp.exp(x - m) + jnp.exp(-m))


def _mlp3(x, p):
    (w1, b1), (w2, b2), (w3, b3) = p
    h = jnp.maximum(jnp.dot(x, w1, preferred_element_type=jnp.float32) + b1, 0.0)
    h = jnp.maximum(jnp.dot(h, w2, preferred_element_type=jnp.float32) + b2, 0.0)
    return jnp.dot(h, w3, preferred_element_type=jnp.float32) + b3


_FUSE_B = 4096
_N_TROWS = 3 * NEAR_L * NEAR_H * NEAR_H


def _fuse_kernel(a_ref, b_ref, o_ref):
    o_ref[:, :NEAR_C] = a_ref[...].astype(jnp.bfloat16)
    o_ref[:, NEAR_C:] = b_ref[...].astype(jnp.bfloat16)


def _fuse_tables(tri_n, tri_s):
    grid = _N_TROWS // _FUSE_B
    return pl.pallas_call(
        _fuse_kernel,
        grid=(grid,),
        in_specs=[pl.BlockSpec((_FUSE_B, NEAR_C), lambda i: (i, 0)),
                  pl.BlockSpec((_FUSE_B, NEAR_C), lambda i: (i, 0))],
        out_specs=pl.BlockSpec((_FUSE_B, 2 * NEAR_C), lambda i: (i, 0)),
        out_shape=jax.ShapeDtypeStruct((_N_TROWS, 2 * NEAR_C), jnp.bfloat16),
    )(tri_n, tri_s)


def _bilinear_corners(u, v, h):
    gx = jnp.clip(u, 0.0, 1.0) * (h - 1)
    gy = jnp.clip(v, 0.0, 1.0) * (h - 1)
    x0f = jnp.floor(gx)
    y0f = jnp.floor(gy)
    x0 = x0f.astype(jnp.int32)
    y0 = y0f.astype(jnp.int32)
    x1 = jnp.minimum(x0 + 1, h - 1)
    y1 = jnp.minimum(y0 + 1, h - 1)
    wx = gx - x0f
    wy = gy - y0f
    return x0, x1, y0, y1, wx, wy


def _mip_levels(r, h, l):
    lvl = jnp.clip(jnp.log(jnp.maximum(r, 1e-6) * h) * INV_LN2, 0.0, l - 1.0)
    l0f = jnp.floor(lvl)
    l0 = l0f.astype(jnp.int32)
    l1 = jnp.minimum(l0 + 1, l - 1)
    wl = lvl - l0f
    return l0, l1, wl


def _corner8(base0, base1, wl, x0, x1, y0, y1, wx, wy, h):
    idxs, ws = [], []
    for lb, lw in ((base0, 1.0 - wl), (base1, wl)):
        for yy, wyy in ((y0, 1.0 - wy), (y1, wy)):
            for xx, wxx in ((x0, 1.0 - wx), (x1, wx)):
                idxs.append(lb + yy * h + xx)
                ws.append(lw * wyy * wxx)
    return idxs, ws


def _prep_rays_kernel(xt_ref, wit_ref, rg_ref, oidx_ref, ow_ref, otab_ref):
    d0 = (wit_ref[0] * 0.5 + 0.5) * 2.0 - 1.0
    d1 = (wit_ref[1] * 0.5 + 0.5) * 2.0 - 1.0
    d2 = (wit_ref[2] * 0.5 + 0.5) * 2.0 - 1.0
    a0, a1, a2 = jnp.abs(d0), jnp.abs(d1), jnp.abs(d2)
    ax0 = (a0 >= a1) & (a0 >= a2)
    ax1 = (~ax0) & (a1 >= a2)
    maj = jnp.where(ax0, d0, jnp.where(ax1, d1, d2))
    face = (jnp.where(ax0, 0, jnp.where(ax1, 2, 4))
            + (maj < 0).astype(jnp.int32))
    su = jnp.where(ax0, d1, d0)
    sv = jnp.where(ax0 | ax1, d2, d1)
    den = jnp.maximum(jnp.abs(maj), 1e-6)
    u = (su / den) * 0.5 + 0.5
    v = (sv / den) * 0.5 + 0.5
    r = rg_ref[0]
    l0, l1, wl = _mip_levels(r, FAR_H, FAR_L)
    x0, x1, y0, y1, wx, wy = _bilinear_corners(u, v, FAR_H)
    fb = face * (FAR_L * FAR_H * FAR_H)
    hh = FAR_H * FAR_H
    idxs, ws = _corner8(fb + l0 * hh, fb + l1 * hh, wl,
                        x0, x1, y0, y1, wx, wy, FAR_H)
    oidx_ref[...] = jnp.concatenate([i[None] for i in idxs], axis=0)
    ow_ref[...] = jnp.concatenate([w[None] for w in ws], axis=0)
    r0 = r * r * math.sqrt(T_CONST / (1.0 - T_CONST))
    zero = jnp.zeros_like(r0)
    rows = [xt_ref[0], xt_ref[1], xt_ref[2],
            wit_ref[0], wit_ref[1], wit_ref[2], r0] + [zero] * 9
    otab_ref[...] = jnp.concatenate([q[None] for q in rows], axis=0)


def _prep_rays(xt, wit, rg):
    return pl.pallas_call(
        _prep_rays_kernel,
        out_shape=(jax.ShapeDtypeStruct((8, N_RAYS), jnp.int32),
                   jax.ShapeDtypeStruct((8, N_RAYS), jnp.float32),
                   jax.ShapeDtypeStruct((16, N_RAYS), jnp.float32)),
    )(xt, wit, rg)


def _sc_gather1(ray_tab, ray_idx2d, cub_idx2d, cub_tab):
    mesh = plsc.VectorSubcoreMesh(core_axis_name="c", subcore_axis_name="s",
                                  num_cores=NC, num_subcores=NS)
    n_attr_rows = N_SAMPLES // NW
    n_cub_rows = N_RAYS * 8 // NW
    CH = 2048
    G = CH // 128

    @functools.partial(
        pl.kernel, mesh=mesh,
        compiler_params=pltpu.CompilerParams(use_tc_tiling_on_sc=False),
        out_type=(jax.ShapeDtypeStruct((N_SAMPLES, 16), jnp.float32),
                  jax.ShapeDtypeStruct((N_RAYS * 8, FAR_C), jnp.float32)),
        scratch_types=[pltpu.VMEM((G, 128), jnp.int32),
                       pltpu.VMEM((CH, 16), jnp.float32),
                       pltpu.VMEM((CH, FAR_C), jnp.float32),
                       pltpu.SemaphoreType.DMA],
    )
    def k(tab_hbm, ridx_hbm, cidx_hbm, ctab_hbm, attrs_hbm, crows_hbm,
          idx_v, rows_v, crows_v, sem):
        wid = lax.axis_index("s") * NC + lax.axis_index("c")

        @pl.loop(0, n_attr_rows // CH)
        def _(i):
            base = wid * n_attr_rows + i * CH
            pltpu.sync_copy(
                ridx_hbm.at[pl.ds(wid * (n_attr_rows // 128) + i * G, G)],
                idx_v)
            for g in range(G):
                pltpu.async_copy(
                    tab_hbm.at[idx_v.at[g]],
                    rows_v.at[pl.ds(g * 128, 128)], sem)
            for g in range(G):
                pltpu.make_async_copy(
                    tab_hbm.at[idx_v.at[g]],
                    rows_v.at[pl.ds(g * 128, 128)], sem).wait()
            pltpu.sync_copy(rows_v, attrs_hbm.at[pl.ds(base, CH)])

        @pl.loop(0, n_cub_rows // CH)
        def _(i):
            base = wid * n_cub_rows + i * CH
            pltpu.sync_copy(
                cidx_hbm.at[pl.ds(wid * (n_cub_rows // 128) + i * G, G)],
                idx_v)
            for g in range(G):
                pltpu.async_copy(
                    ctab_hbm.at[idx_v.at[g]],
                    crows_v.at[pl.ds(g * 128, 128)], sem)
            for g in range(G):
                pltpu.make_async_copy(
                    ctab_hbm.at[idx_v.at[g]],
                    crows_v.at[pl.ds(g * 128, 128)], sem).wait()
            pltpu.sync_copy(crows_v, crows_hbm.at[pl.ds(base, CH)])

    return k(ray_tab, ray_idx2d, cub_idx2d, cub_tab)


_SP_R = 32
_SP_BLK = _SP_R * 128
_SP_GRID = N_SAMPLES // _SP_BLK


def _prep_samples_kernel(at_ref, ts_ref, te_ref, oidx_ref, ow_ref, ov_ref):
    at = at_ref[...]
    ts, te = ts_ref[...], te_ref[...]
    tm = 0.5 * (ts + te)
    xn = [(at[j] + tm * at[3 + j] + 1.0) * 0.5 for j in range(3)]
    rn = at[6] * tm * 0.5
    valid = jnp.ones_like(tm)
    for q in xn:
        valid = valid * ((q >= 0.0) & (q <= 1.0)).astype(jnp.float32)
    ov_ref[...] = valid
    l0, l1, wl = _mip_levels(rn, NEAR_H, NEAR_L)
    hh = NEAR_H * NEAR_H
    idxs_all, ws_all = [], []
    for p, (a, b) in enumerate(((0, 1), (0, 2), (1, 2))):
        x0, x1, y0, y1, wx, wy = _bilinear_corners(xn[a], xn[b], NEAR_H)
        pb = p * NEAR_L * hh
        idxs, ws = _corner8(pb + l0 * hh, pb + l1 * hh, wl,
                            x0, x1, y0, y1, wx, wy, NEAR_H)
        idxs_all += idxs
        ws_all += ws
    oidx_ref[...] = jnp.concatenate([q[None] for q in idxs_all], axis=0)
    ow_ref[...] = jnp.concatenate([q[None] for q in ws_all], axis=0)


def _prep_samples(attrs_t, tsr, ter):
    return pl.pallas_call(
        _prep_samples_kernel,
        grid=(_SP_GRID,),
        in_specs=[pl.BlockSpec((16, _SP_R, 128), lambda i: (0, i, 0)),
                  pl.BlockSpec((_SP_R, 128), lambda i: (i, 0)),
                  pl.BlockSpec((_SP_R, 128), lambda i: (i, 0))],
        out_specs=(pl.BlockSpec((24, _SP_R, 128), lambda i: (0, i, 0)),
                   pl.BlockSpec((24, _SP_R, 128), lambda i: (0, i, 0)),
                   pl.BlockSpec((_SP_R, 128), lambda i: (i, 0))),
        out_shape=(jax.ShapeDtypeStruct((24, N_SAMPLES // 128, 128), jnp.int32),
                   jax.ShapeDtypeStruct((24, N_SAMPLES // 128, 128),
                                        jnp.float32),
                   jax.ShapeDtypeStruct((N_SAMPLES // 128, 128), jnp.float32)),
    )(attrs_t, tsr, ter)


_NG_ROWS = N_SAMPLES * 24


def _sc_gather2(fused, idx2d):
    mesh = plsc.VectorSubcoreMesh(core_axis_name="c", subcore_axis_name="s",
                                  num_cores=NC, num_subcores=NS)
    per_w = _NG_ROWS // NW
    CH = 1024
    G = CH // 128

    @functools.partial(
        pl.kernel, mesh=mesh,
        compiler_params=pltpu.CompilerParams(use_tc_tiling_on_sc=False),
        out_type=jax.ShapeDtypeStruct((_NG_ROWS, 2 * NEAR_C), jnp.bfloat16),
        scratch_types=[pltpu.VMEM((G, 128), jnp.int32),
                       pltpu.VMEM((CH, 2 * NEAR_C), jnp.bfloat16),
                       pltpu.VMEM((G, 128), jnp.int32),
                       pltpu.VMEM((CH, 2 * NEAR_C), jnp.bfloat16),
                       pltpu.SemaphoreType.DMA,
                       pltpu.SemaphoreType.DMA],
    )
    def k(tab_hbm, idx_hbm, g_hbm, idx_a, rows_a, idx_b, rows_b, sem_a, sem_b):
        wid = lax.axis_index("s") * NC + lax.axis_index("c")
        base_w = wid * per_w

        def fire(idx_v, rows_v, sem, i):
            pltpu.sync_copy(
                idx_hbm.at[pl.ds(wid * (per_w // 128) + i * G, G)], idx_v)
            for g in range(G):
                pltpu.async_copy(tab_hbm.at[idx_v.at[g]],
                                 rows_v.at[pl.ds(g * 128, 128)], sem)

        def drain_store(idx_v, rows_v, sem, i):
            base = base_w + i * CH
            for g in range(G):
                pltpu.make_async_copy(tab_hbm.at[idx_v.at[g]],
                                      rows_v.at[pl.ds(g * 128, 128)],
                                      sem).wait()
            pltpu.sync_copy(rows_v, g_hbm.at[pl.ds(base, CH)])

        n_ch = per_w // CH
        fire(idx_a, rows_a, sem_a, 0)

        @pl.loop(0, n_ch // 2)
        def _(j):
            fire(idx_b, rows_b, sem_b, 2 * j + 1)
            drain_store(idx_a, rows_a, sem_a, 2 * j)

            @pl.when(2 * j + 2 < n_ch)
            def _():
                fire(idx_a, rows_a, sem_a, 2 * j + 2)
            drain_store(idx_b, rows_b, sem_b, 2 * j + 1)

    return k(fused, idx2d)


_MB = 2048
_M_GRID = N_SAMPLES // _MB


def _main_kernel(g_ref, w_ref, v_ref,
                 ws1_ref, bs1_ref, ws2_ref, bs2_ref, ws3_ref, bs3_ref,
                 wn1_ref, bn1_ref, wn2_ref, bn2_ref, wn3_ref, bn3_ref,
                 sig_ref, hn_ref):
    g = g_ref[...].astype(jnp.float32)
    wcols = w_ref[...]
    sums = []
    for p in range(3):
        acc = g[8 * p] * wcols[:, 8 * p:8 * p + 1]
        for c in range(1, 8):
            acc = acc + g[8 * p + c] * wcols[:, 8 * p + c:8 * p + c + 1]
        sums.append(acc)
    s0, s1, s2 = sums
    feats_n = jnp.concatenate(
        [s0[:, :NEAR_C], s1[:, :NEAR_C], s2[:, :NEAR_C]], axis=1)
    feats_s = jnp.concatenate(
        [s0[:, NEAR_C:], s1[:, NEAR_C:], s2[:, NEAR_C:]], axis=1)
    ps = ((ws1_ref[...], bs1_ref[...]), (ws2_ref[...], bs2_ref[...]),
          (ws3_ref[...], bs3_ref[...]))
    pn = ((wn1_ref[...], bn1_ref[...]), (wn2_ref[...], bn2_ref[...]),
          (wn3_ref[...], bn3_ref[...]))
    sig = _mlp3(feats_s, ps)
    sig_ref[...] = _softplus(sig) * v_ref[...]
    hn_ref[...] = _mlp3(feats_n, pn)


def _tc_main(g3, w3, valid_col, p_s, p_n):
    full = lambda s: pl.BlockSpec(s, lambda i: tuple(0 for _ in s))
    params = []
    specs = []
    for (w1, b1), (w2, b2), (w3_, b3) in (p_s, p_n):
        for arr in (w1, b1.reshape(1, -1), w2, b2.reshape(1, -1),
                    w3_, b3.reshape(1, -1)):
            params.append(arr)
            specs.append(full(arr.shape))
    return pl.pallas_call(
        _main_kernel,
        grid=(_M_GRID,),
        in_specs=[pl.BlockSpec((24, _MB, 2 * NEAR_C), lambda i: (0, i, 0)),
                  pl.BlockSpec((_MB, 24), lambda i: (i, 0)),
                  pl.BlockSpec((_MB, 1), lambda i: (i, 0))] + specs,
        out_specs=(pl.BlockSpec((_MB, 1), lambda i: (i, 0)),
                   pl.BlockSpec((_MB, FAR_C), lambda i: (i, 0))),
        out_shape=(jax.ShapeDtypeStruct((N_SAMPLES, 1), jnp.float32),
                   jax.ShapeDtypeStruct((N_SAMPLES, FAR_C), jnp.float32)),
    )(g3, w3, valid_col, *params)


_SC_R = N_SAMPLES // 128


def _shift_lanes(y, s, fill=0.0):
    pad = jnp.full((y.shape[0], s), fill, y.dtype)
    return jnp.concatenate([pad, y[:, :-s]], axis=1)


def _shift_rows(y, s, fill=0.0):
    pad = jnp.full((s, y.shape[1]), fill, y.dtype)
    return jnp.concatenate([pad, y[:-s]], axis=0)


def _flat_scan(x, op):
    c = x
    s = 1
    while s < 128:
        c = op(c, _shift_lanes(c, s))
        s *= 2
    t = jnp.broadcast_to(c[:, 127:128], c.shape)
    s = 1
    while s < _SC_R:
        t = op(t, _shift_rows(t, s))
        s *= 2
    return op(c, _shift_rows(t, 1))


def _scan_kernel(sig_ref, ts_ref, te_ref, ray_ref, w_ref):
    sig = sig_ref[...]
    dt = te_ref[...] - ts_ref[...]
    sdt = sig * dt
    cum = _flat_scan(sdt, jnp.add)
    excl = cum - sdt
    ray = ray_ref[...]
    prev = _shift_lanes(ray, 1, 0)
    prev_row = _shift_rows(ray[:, 127:128], 1, -1)
    lane0 = lax.broadcasted_iota(jnp.int32, ray.shape, 1) == 0
    prev = jnp.where(lane0, jnp.broadcast_to(prev_row, ray.shape), prev)
    is_start = ray != prev
    m = jnp.where(is_start, excl, 0.0)
    seg_first = _flat_scan(m, jnp.maximum)
    alpha = 1.0 - jnp.exp(-sdt)
    trans = jnp.exp(-(excl - seg_first))
    w_ref[...] = trans * alpha


def _tc_scan(sig_r, tsr, ter, rayr):
    return pl.pallas_call(
        _scan_kernel,
        out_shape=jax.ShapeDtypeStruct((_SC_R, 128), jnp.float32),
    )(sig_r, tsr, ter, rayr)


def _rows_kernel(hn_ref, w_ref, o_ref):
    w = w_ref[...]
    o_ref[...] = jnp.concatenate(
        [hn_ref[...] * w, w, jnp.zeros((w.shape[0], 15), jnp.float32)], axis=1)


def _tc_rows(h_n, w_col):
    return pl.pallas_call(
        _rows_kernel,
        grid=(_M_GRID,),
        in_specs=[pl.BlockSpec((_MB, FAR_C), lambda i: (i, 0)),
                  pl.BlockSpec((_MB, 1), lambda i: (i, 0))],
        out_specs=pl.BlockSpec((_MB, 48), lambda i: (i, 0)),
        out_shape=jax.ShapeDtypeStruct((N_SAMPLES, 48), jnp.float32),
    )(h_n, w_col)


def _sc_scatter(rows, ray_idx2d, zeros_init):
    mesh = plsc.VectorSubcoreMesh(core_axis_name="c", subcore_axis_name="s",
                                  num_cores=NC, num_subcores=NS)
    per_w = N_SAMPLES // NW
    CH = 1024
    G = CH // 128
    rows_per_tile = N_RAYS // NS

    @functools.partial(
        pl.kernel, mesh=mesh,
        compiler_params=pltpu.CompilerParams(use_tc_tiling_on_sc=False),
        out_type=jax.ShapeDtypeStruct((NC, N_RAYS, 48), jnp.float32),
        scratch_types=[pltpu.VMEM((G, 128), jnp.int32),
                       pltpu.VMEM((CH, 48), jnp.float32),
                       pltpu.VMEM_SHARED((N_RAYS, 48), jnp.float32)],
    )
    def k(rows_hbm, ridx_hbm, zer_hbm, out_hbm, idx_v, rows_v, acc_sh):
        cid = lax.axis_index("c")
        sid = lax.axis_index("s")
        pltpu.sync_copy(zer_hbm.at[pl.ds(sid * rows_per_tile, rows_per_tile)],
                        acc_sh.at[pl.ds(sid * rows_per_tile, rows_per_tile)])
        plsc.subcore_barrier()
        wid = sid * NC + cid

        @pl.loop(0, per_w // CH)
        def _(i):
            base = wid * per_w + i * CH
            pltpu.sync_copy(
                ridx_hbm.at[pl.ds(wid * (per_w // 128) + i * G, G)], idx_v)
            pltpu.sync_copy(rows_hbm.at[pl.ds(base, CH)], rows_v)
            for g in range(G):
                pltpu.sync_copy(rows_v.at[pl.ds(g * 128, 128)],
                                acc_sh.at[idx_v.at[g]], add=True)
        plsc.subcore_barrier()
        pltpu.sync_copy(acc_sh.at[pl.ds(sid * rows_per_tile, rows_per_tile)],
                        out_hbm.at[cid, pl.ds(sid * rows_per_tile,
                                              rows_per_tile)])

    return k(rows, ray_idx2d, zeros_init)


_FB = 2048
_F_GRID = N_RAYS // _FB


def _final_kernel(p_ref, cr_ref, cw_ref, fx_ref, wo_ref,
                  w1_ref, b1_ref, w2_ref, b2_ref, w3_ref, b3_ref, o_ref):
    p = p_ref[...]
    acc = p[0] + p[1]
    h_acc = acc[:, :FAR_C]
    alpha_n = acc[:, FAR_C:FAR_C + 1]
    cw = cw_ref[...]
    h_f = jnp.zeros((_FB, FAR_C), jnp.float32)
    for j in range(8):
        h_f = h_f + cw[:, j:j + 1] * cr_ref[j]
    h = h_f * (1.0 - alpha_n) + h_acc
    inp = jnp.concatenate([fx_ref[...], h, wo_ref[...]], axis=1)
    pd = ((w1_ref[...], b1_ref[...]), (w2_ref[...], b2_ref[...]),
          (w3_ref[...], b3_ref[...]))
    out = _mlp3(inp, pd)
    o_ref[...] = 1.0 / (1.0 + jnp.exp(-out))


def _tc_final(partials, cub_rows3, cub_w, fx, wo_o_n, p_d):
    full = lambda s: pl.BlockSpec(s, lambda i: tuple(0 for _ in s))
    params, specs = [], []
    for w_, b_ in p_d:
        for arr in (w_, b_.reshape(1, -1)):
            params.append(arr)
            specs.append(full(arr.shape))
    return pl.pallas_call(
        _final_kernel,
        grid=(_F_GRID,),
        in_specs=[pl.BlockSpec((NC, _FB, 48), lambda i: (0, i, 0)),
                  pl.BlockSpec((8, _FB, FAR_C), lambda i: (0, i, 0)),
                  pl.BlockSpec((_FB, 8), lambda i: (i, 0)),
                  pl.BlockSpec((_FB, CX), lambda i: (i, 0)),
                  pl.BlockSpec((_FB, 1), lambda i: (i, 0))] + specs,
        out_specs=pl.BlockSpec((_FB, 3), lambda i: (i, 0)),
        out_shape=jax.ShapeDtypeStruct((N_RAYS, 3), jnp.float32),
    )(partials, cub_rows3, cub_w, fx, wo_o_n, *params)


def kernel(x, wi, roughness, fx, wo_o_n, ray_indices, t_starts, t_ends,
           params):
    cub_tab = params['cubemap'].reshape(6 * FAR_L * FAR_H * FAR_H, FAR_C)
    tri_n = params['tri_n'].reshape(_N_TROWS, NEAR_C)
    tri_s = params['tri_n_sigma'].reshape(_N_TROWS, NEAR_C)
    fused = _fuse_tables(tri_n, tri_s)

    cub_idx, cub_w, ray_tab = _prep_rays(
        x.T, wi.T, roughness.reshape(1, N_RAYS))
    ray_tab_r = ray_tab.T.reshape(N_RAYS, 16)
    cub_idx_flat = cub_idx.reshape(N_RAYS * 8 // 128, 128)
    cub_w_r = cub_w.T

    ray_idx2d = ray_indices.reshape(N_SAMPLES // 128, 128)
    attrs, cub_rows = _sc_gather1(ray_tab_r, ray_idx2d, cub_idx_flat, cub_tab)

    attrs_t = attrs.T.reshape(16, N_SAMPLES // 128, 128)
    tsr = t_starts.reshape(N_SAMPLES // 128, 128)
    ter = t_ends.reshape(N_SAMPLES // 128, 128)
    idx24, w24, valid = _prep_samples(attrs_t, tsr, ter)

    idx_flat = idx24.reshape(_NG_ROWS // 128, 128)
    g = _sc_gather2(fused, idx_flat)

    g3 = g.reshape(24, N_SAMPLES, 2 * NEAR_C)
    w_cols = w24.reshape(24, N_SAMPLES).T
    valid_col = valid.reshape(N_SAMPLES, 1)
    sig, h_n = _tc_main(g3, w_cols, valid_col,
                        params['mlp_n_sigma'], params['mlp_n'])

    w_flat = _tc_scan(sig.reshape(N_SAMPLES // 128, 128), tsr, ter,
                      ray_indices.reshape(N_SAMPLES // 128, 128))
    rows = _tc_rows(h_n, w_flat.reshape(N_SAMPLES, 1))

    zeros_init = jnp.zeros((N_RAYS, 48), jnp.float32)
    partials = _sc_scatter(rows, ray_idx2d, zeros_init)

    cub_rows3 = cub_rows.reshape(8, N_RAYS, FAR_C)
    return _tc_final(partials, cub_rows3, cub_w_r, fx, wo_o_n,
                     params['mlp_d'])


# speedup vs baseline: 1.5902x; 1.0067x over previous
import functools
import math

import jax
import jax.numpy as jnp
from jax import lax
from jax.experimental import pallas as pl
from jax.experimental.pallas import tpu as pltpu
from jax.experimental.pallas import tpu_sc as plsc

N_RAYS = 16384
N_SAMPLES = 524288
CX = 64
FAR_C, FAR_H, FAR_L = 32, 128, 4
NEAR_C, NEAR_H, NEAR_L = 16, 256, 4
T_CONST = 0.75
INV_LN2 = 1.4426950408889634

NC, NS = 2, 16
NW = NC * NS


def _softplus(x):
    m = jnp.maximum(x, 0.0)
    return m + jnp.log(jnp.exp(x - m) + jnp.exp(-m))


def _mlp3(x, p):
    (w1, b1), (w2, b2), (w3, b3) = p
    h = jnp.maximum(jnp.dot(x, w1, preferred_element_type=jnp.float32) + b1, 0.0)
    h = jnp.maximum(jnp.dot(h, w2, preferred_element_type=jnp.float32) + b2, 0.0)
    return jnp.dot(h, w3, preferred_element_type=jnp.float32) + b3


_FUSE_B = 16384
_N_TROWS = 3 * NEAR_L * NEAR_H * NEAR_H


def _fuse_kernel(a_ref, b_ref, o_ref):
    o_ref[:, :NEAR_C] = a_ref[...].astype(jnp.bfloat16)
    o_ref[:, NEAR_C:] = b_ref[...].astype(jnp.bfloat16)


def _fuse_tables(tri_n, tri_s):
    grid = _N_TROWS // _FUSE_B
    return pl.pallas_call(
        _fuse_kernel,
        grid=(grid,),
        in_specs=[pl.BlockSpec((_FUSE_B, NEAR_C), lambda i: (i, 0)),
                  pl.BlockSpec((_FUSE_B, NEAR_C), lambda i: (i, 0))],
        out_specs=pl.BlockSpec((_FUSE_B, 2 * NEAR_C), lambda i: (i, 0)),
        out_shape=jax.ShapeDtypeStruct((_N_TROWS, 2 * NEAR_C), jnp.bfloat16),
    )(tri_n, tri_s)


def _bilinear_corners(u, v, h):
    gx = jnp.clip(u, 0.0, 1.0) * (h - 1)
    gy = jnp.clip(v, 0.0, 1.0) * (h - 1)
    x0f = jnp.floor(gx)
    y0f = jnp.floor(gy)
    x0 = x0f.astype(jnp.int32)
    y0 = y0f.astype(jnp.int32)
    x1 = jnp.minimum(x0 + 1, h - 1)
    y1 = jnp.minimum(y0 + 1, h - 1)
    wx = gx - x0f
    wy = gy - y0f
    return x0, x1, y0, y1, wx, wy


def _mip_levels(r, h, l):
    lvl = jnp.clip(jnp.log(jnp.maximum(r, 1e-6) * h) * INV_LN2, 0.0, l - 1.0)
    l0f = jnp.floor(lvl)
    l0 = l0f.astype(jnp.int32)
    l1 = jnp.minimum(l0 + 1, l - 1)
    wl = lvl - l0f
    return l0, l1, wl


def _corner8(base0, base1, wl, x0, x1, y0, y1, wx, wy, h):
    idxs, ws = [], []
    for lb, lw in ((base0, 1.0 - wl), (base1, wl)):
        for yy, wyy in ((y0, 1.0 - wy), (y1, wy)):
            for xx, wxx in ((x0, 1.0 - wx), (x1, wx)):
                idxs.append(lb + yy * h + xx)
                ws.append(lw * wyy * wxx)
    return idxs, ws


def _prep_rays_kernel(xt_ref, wit_ref, rg_ref, oidx_ref, ow_ref, otab_ref):
    d0 = (wit_ref[0] * 0.5 + 0.5) * 2.0 - 1.0
    d1 = (wit_ref[1] * 0.5 + 0.5) * 2.0 - 1.0
    d2 = (wit_ref[2] * 0.5 + 0.5) * 2.0 - 1.0
    a0, a1, a2 = jnp.abs(d0), jnp.abs(d1), jnp.abs(d2)
    ax0 = (a0 >= a1) & (a0 >= a2)
    ax1 = (~ax0) & (a1 >= a2)
    maj = jnp.where(ax0, d0, jnp.where(ax1, d1, d2))
    face = (jnp.where(ax0, 0, jnp.where(ax1, 2, 4))
            + (maj < 0).astype(jnp.int32))
    su = jnp.where(ax0, d1, d0)
    sv = jnp.where(ax0 | ax1, d2, d1)
    den = jnp.maximum(jnp.abs(maj), 1e-6)
    u = (su / den) * 0.5 + 0.5
    v = (sv / den) * 0.5 + 0.5
    r = rg_ref[0]
    l0, l1, wl = _mip_levels(r, FAR_H, FAR_L)
    x0, x1, y0, y1, wx, wy = _bilinear_corners(u, v, FAR_H)
    fb = face * (FAR_L * FAR_H * FAR_H)
    hh = FAR_H * FAR_H
    idxs, ws = _corner8(fb + l0 * hh, fb + l1 * hh, wl,
                        x0, x1, y0, y1, wx, wy, FAR_H)
    oidx_ref[...] = jnp.concatenate([i[None] for i in idxs], axis=0)
    ow_ref[...] = jnp.concatenate([w[None] for w in ws], axis=0)
    r0 = r * r * math.sqrt(T_CONST / (1.0 - T_CONST))
    zero = jnp.zeros_like(r0)
    rows = [xt_ref[0], xt_ref[1], xt_ref[2],
            wit_ref[0], wit_ref[1], wit_ref[2], r0] + [zero] * 9
    otab_ref[...] = jnp.concatenate([q[None] for q in rows], axis=0)


def _prep_rays(xt, wit, rg):
    return pl.pallas_call(
        _prep_rays_kernel,
        out_shape=(jax.ShapeDtypeStruct((8, N_RAYS), jnp.int32),
                   jax.ShapeDtypeStruct((8, N_RAYS), jnp.float32),
                   jax.ShapeDtypeStruct((16, N_RAYS), jnp.float32)),
    )(xt, wit, rg)


def _sc_gather1(ray_tab, ray_idx2d, cub_idx2d, cub_tab):
    mesh = plsc.VectorSubcoreMesh(core_axis_name="c", subcore_axis_name="s",
                                  num_cores=NC, num_subcores=NS)
    n_attr_rows = N_SAMPLES // NW
    n_cub_rows = N_RAYS * 8 // NW
    CH = 2048
    G = CH // 128

    @functools.partial(
        pl.kernel, mesh=mesh,
        compiler_params=pltpu.CompilerParams(use_tc_tiling_on_sc=False),
        out_type=(jax.ShapeDtypeStruct((N_SAMPLES, 16), jnp.float32),
                  jax.ShapeDtypeStruct((N_RAYS * 8, FAR_C), jnp.float32)),
        scratch_types=[pltpu.VMEM((G, 128), jnp.int32),
                       pltpu.VMEM((CH, 16), jnp.float32),
                       pltpu.VMEM((CH, FAR_C), jnp.float32),
                       pltpu.SemaphoreType.DMA],
    )
    def k(tab_hbm, ridx_hbm, cidx_hbm, ctab_hbm, attrs_hbm, crows_hbm,
          idx_v, rows_v, crows_v, sem):
        wid = lax.axis_index("s") * NC + lax.axis_index("c")

        @pl.loop(0, n_attr_rows // CH)
        def _(i):
            base = wid * n_attr_rows + i * CH
            pltpu.sync_copy(
                ridx_hbm.at[pl.ds(wid * (n_attr_rows // 128) + i * G, G)],
                idx_v)
            for g in range(G):
                pltpu.async_copy(
                    tab_hbm.at[idx_v.at[g]],
                    rows_v.at[pl.ds(g * 128, 128)], sem)
            for g in range(G):
                pltpu.make_async_copy(
                    tab_hbm.at[idx_v.at[g]],
                    rows_v.at[pl.ds(g * 128, 128)], sem).wait()
            pltpu.sync_copy(rows_v, attrs_hbm.at[pl.ds(base, CH)])

        @pl.loop(0, n_cub_rows // CH)
        def _(i):
            base = wid * n_cub_rows + i * CH
            pltpu.sync_copy(
                cidx_hbm.at[pl.ds(wid * (n_cub_rows // 128) + i * G, G)],
                idx_v)
            for g in range(G):
                pltpu.async_copy(
                    ctab_hbm.at[idx_v.at[g]],
                    crows_v.at[pl.ds(g * 128, 128)], sem)
            for g in range(G):
                pltpu.make_async_copy(
                    ctab_hbm.at[idx_v.at[g]],
                    crows_v.at[pl.ds(g * 128, 128)], sem).wait()
            pltpu.sync_copy(crows_v, crows_hbm.at[pl.ds(base, CH)])

    return k(ray_tab, ray_idx2d, cub_idx2d, cub_tab)


_SP_R = 128
_SP_BLK = _SP_R * 128
_SP_GRID = N_SAMPLES // _SP_BLK


def _prep_samples_kernel(at_ref, ts_ref, te_ref, oidx_ref, ow_ref, ov_ref):
    at = at_ref[...]
    ts, te = ts_ref[...], te_ref[...]
    tm = 0.5 * (ts + te)
    xn = [(at[j] + tm * at[3 + j] + 1.0) * 0.5 for j in range(3)]
    rn = at[6] * tm * 0.5
    valid = jnp.ones_like(tm)
    for q in xn:
        valid = valid * ((q >= 0.0) & (q <= 1.0)).astype(jnp.float32)
    ov_ref[...] = valid
    l0, l1, wl = _mip_levels(rn, NEAR_H, NEAR_L)
    hh = NEAR_H * NEAR_H
    idxs_all, ws_all = [], []
    for p, (a, b) in enumerate(((0, 1), (0, 2), (1, 2))):
        x0, x1, y0, y1, wx, wy = _bilinear_corners(xn[a], xn[b], NEAR_H)
        pb = p * NEAR_L * hh
        idxs, ws = _corner8(pb + l0 * hh, pb + l1 * hh, wl,
                            x0, x1, y0, y1, wx, wy, NEAR_H)
        idxs_all += idxs
        ws_all += ws
    oidx_ref[...] = jnp.concatenate([q[None] for q in idxs_all], axis=0)
    ow_ref[...] = jnp.concatenate([q[None] for q in ws_all], axis=0)


def _prep_samples(attrs_t, tsr, ter):
    return pl.pallas_call(
        _prep_samples_kernel,
        grid=(_SP_GRID,),
        in_specs=[pl.BlockSpec((16, _SP_R, 128), lambda i: (0, i, 0)),
                  pl.BlockSpec((_SP_R, 128), lambda i: (i, 0)),
                  pl.BlockSpec((_SP_R, 128), lambda i: (i, 0))],
        out_specs=(pl.BlockSpec((24, _SP_R, 128), lambda i: (0, i, 0)),
                   pl.BlockSpec((24, _SP_R, 128), lambda i: (0, i, 0)),
                   pl.BlockSpec((_SP_R, 128), lambda i: (i, 0))),
        out_shape=(jax.ShapeDtypeStruct((24, N_SAMPLES // 128, 128), jnp.int32),
                   jax.ShapeDtypeStruct((24, N_SAMPLES // 128, 128),
                                        jnp.float32),
                   jax.ShapeDtypeStruct((N_SAMPLES // 128, 128), jnp.float32)),
    )(attrs_t, tsr, ter)


_NG_ROWS = N_SAMPLES * 24


def _sc_gather2(fused, idx2d):
    mesh = plsc.VectorSubcoreMesh(core_axis_name="c", subcore_axis_name="s",
                                  num_cores=NC, num_subcores=NS)
    per_w = _NG_ROWS // NW
    CH = 1024
    G = CH // 128

    @functools.partial(
        pl.kernel, mesh=mesh,
        compiler_params=pltpu.CompilerParams(use_tc_tiling_on_sc=False),
        out_type=jax.ShapeDtypeStruct((_NG_ROWS, 2 * NEAR_C), jnp.bfloat16),
        scratch_types=[pltpu.VMEM((G, 128), jnp.int32),
                       pltpu.VMEM((CH, 2 * NEAR_C), jnp.bfloat16),
                       pltpu.VMEM((G, 128), jnp.int32),
                       pltpu.VMEM((CH, 2 * NEAR_C), jnp.bfloat16),
                       pltpu.SemaphoreType.DMA,
                       pltpu.SemaphoreType.DMA],
    )
    def k(tab_hbm, idx_hbm, g_hbm, idx_a, rows_a, idx_b, rows_b, sem_a, sem_b):
        wid = lax.axis_index("s") * NC + lax.axis_index("c")
        base_w = wid * per_w

        def fire(idx_v, rows_v, sem, i):
            pltpu.sync_copy(
                idx_hbm.at[pl.ds(wid * (per_w // 128) + i * G, G)], idx_v)
            for g in range(G):
                pltpu.async_copy(tab_hbm.at[idx_v.at[g]],
                                 rows_v.at[pl.ds(g * 128, 128)], sem)

        def drain_store(idx_v, rows_v, sem, i):
            base = base_w + i * CH
            for g in range(G):
                pltpu.make_async_copy(tab_hbm.at[idx_v.at[g]],
                                      rows_v.at[pl.ds(g * 128, 128)],
                                      sem).wait()
            pltpu.sync_copy(rows_v, g_hbm.at[pl.ds(base, CH)])

        n_ch = per_w // CH
        fire(idx_a, rows_a, sem_a, 0)

        @pl.loop(0, n_ch // 2)
        def _(j):
            fire(idx_b, rows_b, sem_b, 2 * j + 1)
            drain_store(idx_a, rows_a, sem_a, 2 * j)

            @pl.when(2 * j + 2 < n_ch)
            def _():
                fire(idx_a, rows_a, sem_a, 2 * j + 2)
            drain_store(idx_b, rows_b, sem_b, 2 * j + 1)

    return k(fused, idx2d)


_MB = 2048
_M_GRID = N_SAMPLES // _MB


def _main_kernel(g_ref, w_ref, v_ref,
                 ws1_ref, bs1_ref, ws2_ref, bs2_ref, ws3_ref, bs3_ref,
                 wn1_ref, bn1_ref, wn2_ref, bn2_ref, wn3_ref, bn3_ref,
                 sig_ref, hn_ref):
    g = g_ref[...].astype(jnp.float32)
    wcols = w_ref[...]
    sums = []
    for p in range(3):
        acc = g[8 * p] * wcols[:, 8 * p:8 * p + 1]
        for c in range(1, 8):
            acc = acc + g[8 * p + c] * wcols[:, 8 * p + c:8 * p + c + 1]
        sums.append(acc)
    s0, s1, s2 = sums
    feats_n = jnp.concatenate(
        [s0[:, :NEAR_C], s1[:, :NEAR_C], s2[:, :NEAR_C]], axis=1)
    feats_s = jnp.concatenate(
        [s0[:, NEAR_C:], s1[:, NEAR_C:], s2[:, NEAR_C:]], axis=1)
    ps = ((ws1_ref[...], bs1_ref[...]), (ws2_ref[...], bs2_ref[...]),
          (ws3_ref[...], bs3_ref[...]))
    pn = ((wn1_ref[...], bn1_ref[...]), (wn2_ref[...], bn2_ref[...]),
          (wn3_ref[...], bn3_ref[...]))
    sig = _mlp3(feats_s, ps)
    sig_ref[...] = _softplus(sig) * v_ref[...]
    hn_ref[...] = _mlp3(feats_n, pn)


def _tc_main(g3, w3, valid_col, p_s, p_n):
    full = lambda s: pl.BlockSpec(s, lambda i: tuple(0 for _ in s))
    params = []
    specs = []
    for (w1, b1), (w2, b2), (w3_, b3) in (p_s, p_n):
        for arr in (w1, b1.reshape(1, -1), w2, b2.reshape(1, -1),
                    w3_, b3.reshape(1, -1)):
            params.append(arr)
            specs.append(full(arr.shape))
    return pl.pallas_call(
        _main_kernel,
        grid=(_M_GRID,),
        in_specs=[pl.BlockSpec((24, _MB, 2 * NEAR_C), lambda i: (0, i, 0)),
                  pl.BlockSpec((_MB, 24), lambda i: (i, 0)),
                  pl.BlockSpec((_MB, 1), lambda i: (i, 0))] + specs,
        out_specs=(pl.BlockSpec((_MB, 1), lambda i: (i, 0)),
                   pl.BlockSpec((_MB, FAR_C), lambda i: (i, 0))),
        out_shape=(jax.ShapeDtypeStruct((N_SAMPLES, 1), jnp.float32),
                   jax.ShapeDtypeStruct((N_SAMPLES, FAR_C), jnp.float32)),
    )(g3, w3, valid_col, *params)


_SC_R = N_SAMPLES // 128


def _shift_lanes(y, s, fill=0.0):
    pad = jnp.full((y.shape[0], s), fill, y.dtype)
    return jnp.concatenate([pad, y[:, :-s]], axis=1)


def _shift_rows(y, s, fill=0.0):
    pad = jnp.full((s, y.shape[1]), fill, y.dtype)
    return jnp.concatenate([pad, y[:-s]], axis=0)


def _flat_scan(x, op):
    c = x
    s = 1
    while s < 128:
        c = op(c, _shift_lanes(c, s))
        s *= 2
    t = jnp.broadcast_to(c[:, 127:128], c.shape)
    s = 1
    while s < _SC_R:
        t = op(t, _shift_rows(t, s))
        s *= 2
    return op(c, _shift_rows(t, 1))


def _scan_kernel(sig_ref, ts_ref, te_ref, ray_ref, w_ref):
    sig = sig_ref[...]
    dt = te_ref[...] - ts_ref[...]
    sdt = sig * dt
    cum = _flat_scan(sdt, jnp.add)
    excl = cum - sdt
    ray = ray_ref[...]
    prev = _shift_lanes(ray, 1, 0)
    prev_row = _shift_rows(ray[:, 127:128], 1, -1)
    lane0 = lax.broadcasted_iota(jnp.int32, ray.shape, 1) == 0
    prev = jnp.where(lane0, jnp.broadcast_to(prev_row, ray.shape), prev)
    is_start = ray != prev
    m = jnp.where(is_start, excl, 0.0)
    seg_first = _flat_scan(m, jnp.maximum)
    alpha = 1.0 - jnp.exp(-sdt)
    trans = jnp.exp(-(excl - seg_first))
    w_ref[...] = trans * alpha


def _tc_scan(sig_r, tsr, ter, rayr):
    return pl.pallas_call(
        _scan_kernel,
        out_shape=jax.ShapeDtypeStruct((_SC_R, 128), jnp.float32),
    )(sig_r, tsr, ter, rayr)


def _rows_kernel(hn_ref, w_ref, o_ref):
    w = w_ref[...]
    o_ref[...] = jnp.concatenate(
        [hn_ref[...] * w, w, jnp.zeros((w.shape[0], 15), jnp.float32)], axis=1)


def _tc_rows(h_n, w_col):
    return pl.pallas_call(
        _rows_kernel,
        grid=(_M_GRID,),
        in_specs=[pl.BlockSpec((_MB, FAR_C), lambda i: (i, 0)),
                  pl.BlockSpec((_MB, 1), lambda i: (i, 0))],
        out_specs=pl.BlockSpec((_MB, 48), lambda i: (i, 0)),
        out_shape=jax.ShapeDtypeStruct((N_SAMPLES, 48), jnp.float32),
    )(h_n, w_col)


def _sc_scatter(rows, ray_idx2d, zeros_init):
    mesh = plsc.VectorSubcoreMesh(core_axis_name="c", subcore_axis_name="s",
                                  num_cores=NC, num_subcores=NS)
    per_w = N_SAMPLES // NW
    CH = 1024
    G = CH // 128
    rows_per_tile = N_RAYS // NS

    @functools.partial(
        pl.kernel, mesh=mesh,
        compiler_params=pltpu.CompilerParams(use_tc_tiling_on_sc=False),
        out_type=jax.ShapeDtypeStruct((NC, N_RAYS, 48), jnp.float32),
        scratch_types=[pltpu.VMEM((G, 128), jnp.int32),
                       pltpu.VMEM((CH, 48), jnp.float32),
                       pltpu.VMEM_SHARED((N_RAYS, 48), jnp.float32)],
    )
    def k(rows_hbm, ridx_hbm, zer_hbm, out_hbm, idx_v, rows_v, acc_sh):
        cid = lax.axis_index("c")
        sid = lax.axis_index("s")
        pltpu.sync_copy(zer_hbm.at[pl.ds(sid * rows_per_tile, rows_per_tile)],
                        acc_sh.at[pl.ds(sid * rows_per_tile, rows_per_tile)])
        plsc.subcore_barrier()
        wid = sid * NC + cid

        @pl.loop(0, per_w // CH)
        def _(i):
            base = wid * per_w + i * CH
            pltpu.sync_copy(
                ridx_hbm.at[pl.ds(wid * (per_w // 128) + i * G, G)], idx_v)
            pltpu.sync_copy(rows_hbm.at[pl.ds(base, CH)], rows_v)
            for g in range(G):
                pltpu.sync_copy(rows_v.at[pl.ds(g * 128, 128)],
                                acc_sh.at[idx_v.at[g]], add=True)
        plsc.subcore_barrier()
        pltpu.sync_copy(acc_sh.at[pl.ds(sid * rows_per_tile, rows_per_tile)],
                        out_hbm.at[cid, pl.ds(sid * rows_per_tile,
                                              rows_per_tile)])

    return k(rows, ray_idx2d, zeros_init)


_FB = 2048
_F_GRID = N_RAYS // _FB


def _final_kernel(p_ref, cr_ref, cw_ref, fx_ref, wo_ref,
                  w1_ref, b1_ref, w2_ref, b2_ref, w3_ref, b3_ref, o_ref):
    p = p_ref[...]
    acc = p[0] + p[1]
    h_acc = acc[:, :FAR_C]
    alpha_n = acc[:, FAR_C:FAR_C + 1]
    cw = cw_ref[...]
    h_f = jnp.zeros((_FB, FAR_C), jnp.float32)
    for j in range(8):
        h_f = h_f + cw[:, j:j + 1] * cr_ref[j]
    h = h_f * (1.0 - alpha_n) + h_acc
    inp = jnp.concatenate([fx_ref[...], h, wo_ref[...]], axis=1)
    pd = ((w1_ref[...], b1_ref[...]), (w2_ref[...], b2_ref[...]),
          (w3_ref[...], b3_ref[...]))
    out = _mlp3(inp, pd)
    o_ref[...] = 1.0 / (1.0 + jnp.exp(-out))


def _tc_final(partials, cub_rows3, cub_w, fx, wo_o_n, p_d):
    full = lambda s: pl.BlockSpec(s, lambda i: tuple(0 for _ in s))
    params, specs = [], []
    for w_, b_ in p_d:
        for arr in (w_, b_.reshape(1, -1)):
            params.append(arr)
            specs.append(full(arr.shape))
    return pl.pallas_call(
        _final_kernel,
        grid=(_F_GRID,),
        in_specs=[pl.BlockSpec((NC, _FB, 48), lambda i: (0, i, 0)),
                  pl.BlockSpec((8, _FB, FAR_C), lambda i: (0, i, 0)),
                  pl.BlockSpec((_FB, 8), lambda i: (i, 0)),
                  pl.BlockSpec((_FB, CX), lambda i: (i, 0)),
                  pl.BlockSpec((_FB, 1), lambda i: (i, 0))] + specs,
        out_specs=pl.BlockSpec((_FB, 3), lambda i: (i, 0)),
        out_shape=jax.ShapeDtypeStruct((N_RAYS, 3), jnp.float32),
    )(partials, cub_rows3, cub_w, fx, wo_o_n, *params)


def kernel(x, wi, roughness, fx, wo_o_n, ray_indices, t_starts, t_ends,
           params):
    cub_tab = params['cubemap'].reshape(6 * FAR_L * FAR_H * FAR_H, FAR_C)
    tri_n = params['tri_n'].reshape(_N_TROWS, NEAR_C)
    tri_s = params['tri_n_sigma'].reshape(_N_TROWS, NEAR_C)
    fused = _fuse_tables(tri_n, tri_s)

    cub_idx, cub_w, ray_tab = _prep_rays(
        x.T, wi.T, roughness.reshape(1, N_RAYS))
    ray_tab_r = ray_tab.T.reshape(N_RAYS, 16)
    cub_idx_flat = cub_idx.reshape(N_RAYS * 8 // 128, 128)
    cub_w_r = cub_w.T

    ray_idx2d = ray_indices.reshape(N_SAMPLES // 128, 128)
    attrs, cub_rows = _sc_gather1(ray_tab_r, ray_idx2d, cub_idx_flat, cub_tab)

    attrs_t = attrs.T.reshape(16, N_SAMPLES // 128, 128)
    tsr = t_starts.reshape(N_SAMPLES // 128, 128)
    ter = t_ends.reshape(N_SAMPLES // 128, 128)
    idx24, w24, valid = _prep_samples(attrs_t, tsr, ter)

    idx_flat = idx24.reshape(_NG_ROWS // 128, 128)
    g = _sc_gather2(fused, idx_flat)

    g3 = g.reshape(24, N_SAMPLES, 2 * NEAR_C)
    w_cols = w24.reshape(24, N_SAMPLES).T
    valid_col = valid.reshape(N_SAMPLES, 1)
    sig, h_n = _tc_main(g3, w_cols, valid_col,
                        params['mlp_n_sigma'], params['mlp_n'])

    w_flat = _tc_scan(sig.reshape(N_SAMPLES // 128, 128), tsr, ter,
                      ray_indices.reshape(N_SAMPLES // 128, 128))
    rows = _tc_rows(h_n, w_flat.reshape(N_SAMPLES, 1))

    zeros_init = jnp.zeros((N_RAYS, 48), jnp.float32)
    partials = _sc_scatter(rows, ray_idx2d, zeros_init)

    cub_rows3 = cub_rows.reshape(8, N_RAYS, FAR_C)
    return _tc_final(partials, cub_rows3, cub_w_r, fx, wo_o_n,
                     params['mlp_d'])
